# baseline (speedup 1.0000x reference)
import jax
import jax.numpy as jnp
from jax import lax
from jax.experimental import pallas as pl
from jax.experimental.pallas import tpu as pltpu
from jax.experimental.pallas import tpu_sc as plsc

_INFO = plsc.get_sparse_core_info()
_NC = _INFO.num_cores
_NS = _INFO.num_subcores
_NL = _INFO.num_lanes
_NW = _NC * _NS


def _make_gather(n_idx, n_rows, dim):
    assert n_idx % (8 * _NW) == 0
    per_w = n_idx // _NW
    mesh = plsc.VectorSubcoreMesh(core_axis_name="c", subcore_axis_name="s")

    @pl.kernel(
        mesh=mesh,
        compiler_params=pltpu.CompilerParams(use_tc_tiling_on_sc=False),
        out_type=(
            jax.ShapeDtypeStruct((n_idx, dim), jnp.float32),
            jax.ShapeDtypeStruct((n_idx, dim), jnp.float32),
        ),
        scratch_types=[
            pltpu.VMEM((per_w,), jnp.int32),
            pltpu.VMEM((per_w,), jnp.int32),
            pltpu.VMEM((per_w, dim), jnp.float32),
            pltpu.VMEM((per_w, dim), jnp.float32),
            pltpu.SemaphoreType.DMA,
            pltpu.SemaphoreType.DMA,
            pltpu.SemaphoreType.DMA,
            pltpu.SemaphoreType.DMA,
        ],
    )
    def gather_kernel(pos_hbm, tab_hbm, cos_out, sin_out,
                      idx_v, idx2_v, cos_v, sin_v,
                      sem_c, sem_s, sem_oc, sem_os):
        wid = lax.axis_index("s") * _NC + lax.axis_index("c")
        base = wid * per_w
        pltpu.sync_copy(pos_hbm.at[pl.ds(base, per_w)], idx_v)
        cpy_c = pltpu.async_copy(tab_hbm.at[idx_v], cos_v, sem_c)
        for i in range(per_w // _NL):
            sl = pl.ds(i * _NL, _NL)
            idx2_v[sl] = idx_v[sl] + n_rows
        cpy_s = pltpu.async_copy(tab_hbm.at[idx2_v], sin_v, sem_s)
        cpy_c.wait()
        out_c = pltpu.async_copy(
            cos_v, cos_out.at[pl.ds(base, per_w)], sem_oc)
        cpy_s.wait()
        out_s = pltpu.async_copy(
            sin_v, sin_out.at[pl.ds(base, per_w)], sem_os)
        out_c.wait()
        out_s.wait()

    return gather_kernel


def kernel(qkv, position_ids, cos, sin):
    b, s = position_ids.shape
    n_rows, dim = cos.shape
    flat_ids = position_ids.reshape(b * s).astype(jnp.int32)
    table = jnp.concatenate([cos, sin], axis=0)
    cos_flat, sin_flat = _make_gather(b * s, n_rows, dim)(flat_ids, table)
    return cos_flat.reshape(b, s, dim), sin_flat.reshape(b, s, dim)


# speedup vs baseline: 1.1650x; 1.1650x over previous
import jax
import jax.numpy as jnp
from jax import lax
from jax.experimental import pallas as pl
from jax.experimental.pallas import tpu as pltpu
from jax.experimental.pallas import tpu_sc as plsc

_INFO = plsc.get_sparse_core_info()
_NC = _INFO.num_cores
_NS = _INFO.num_subcores
_NL = _INFO.num_lanes
_CH = 128


def _make(b, s, n_rows, dim):
    n_idx = b * s
    per_t = n_idx // _NS
    rows_t = n_rows // _NS
    mesh = plsc.VectorSubcoreMesh(core_axis_name="c", subcore_axis_name="s")

    @pl.kernel(
        mesh=mesh,
        compiler_params=pltpu.CompilerParams(use_tc_tiling_on_sc=True),
        out_type=(
            jax.ShapeDtypeStruct((b, s, dim), jnp.float32),
            jax.ShapeDtypeStruct((b, s, dim), jnp.float32),
        ),
        scratch_types=[
            pltpu.VMEM((per_t,), jnp.int32),
            pltpu.VMEM((per_t,), jnp.int32),
            pltpu.VMEM((_CH, dim), jnp.float32),
            pltpu.VMEM((_CH, dim), jnp.float32),
            pltpu.VMEM_SHARED((n_rows, dim), jnp.float32),
            pltpu.SemaphoreType.DMA,
        ],
    )
    def k(pos_hbm, cos_hbm, sin_hbm, cos_out, sin_out,
          idx_v, idx2_v, rows_v, stage_v, tab_sh, sem):
        cid = lax.axis_index("c")
        sid = lax.axis_index("s")
        base = sid * per_t
        bi = base // s
        ri = base % s
        r0 = sid * rows_t

        for j in range(rows_t // _CH):
            @pl.when(cid == 0)
            def _():
                pltpu.sync_copy(cos_hbm.at[pl.ds(r0 + j * _CH, _CH)], stage_v)

            @pl.when(cid == 1)
            def _():
                pltpu.sync_copy(sin_hbm.at[pl.ds(r0 + j * _CH, _CH)], stage_v)

            pltpu.sync_copy(stage_v, tab_sh.at[pl.ds(r0 + j * _CH, _CH)])

        pltpu.sync_copy(pos_hbm.at[bi, pl.ds(ri, per_t)], idx_v)
        for i in range(per_t // _NL):
            sl = pl.ds(i * _NL, _NL)
            idx2_v[sl] = idx_v[sl] * 2
        plsc.subcore_barrier()

        for j in range(per_t // _CH):
            pltpu.async_copy(
                tab_sh.at[idx2_v.at[pl.ds(j * _CH, _CH)]], rows_v, sem).wait()

            @pl.when(cid == 0)
            def _():
                pltpu.sync_copy(
                    rows_v, cos_out.at[bi, pl.ds(ri + j * _CH, _CH)])

            @pl.when(cid == 1)
            def _():
                pltpu.sync_copy(
                    rows_v, sin_out.at[bi, pl.ds(ri + j * _CH, _CH)])

    return k


def kernel(qkv, position_ids, cos, sin):
    b, s = position_ids.shape
    n_rows, dim = cos.shape
    return _make(b, s, n_rows, dim)(position_ids.astype(jnp.int32), cos, sin)


# speedup vs baseline: 1.3962x; 1.1984x over previous
import jax
import jax.numpy as jnp
from jax import lax
from jax.experimental import pallas as pl
from jax.experimental.pallas import tpu as pltpu
from jax.experimental.pallas import tpu_sc as plsc

_INFO = plsc.get_sparse_core_info()
_NC = _INFO.num_cores
_NS = _INFO.num_subcores
_NL = _INFO.num_lanes


def _make(b, s, n_rows, dim):
    groups = _NS // b
    rows_g = dim // groups
    mesh = plsc.VectorSubcoreMesh(core_axis_name="c", subcore_axis_name="s")

    @pl.kernel(
        mesh=mesh,
        compiler_params=pltpu.CompilerParams(use_tc_tiling_on_sc=True,
                                             needs_layout_passes=False),
        out_type=(
            jax.ShapeDtypeStruct((b, dim, s), jnp.float32),
            jax.ShapeDtypeStruct((b, dim, s), jnp.float32),
        ),
        scratch_types=[
            pltpu.VMEM((s,), jnp.int32),
            pltpu.VMEM((rows_g, n_rows), jnp.float32),
            pltpu.VMEM((rows_g, s), jnp.float32),
        ],
    )
    def k(pos_hbm, cos_t_hbm, sin_t_hbm, cos_out, sin_out,
          idx_v, stage_v, out_v):
        cid = lax.axis_index("c")
        sid = lax.axis_index("s")
        g = sid % groups
        h = sid // groups
        d0 = g * rows_g

        pltpu.sync_copy(pos_hbm.at[h], idx_v)

        def pipeline(tab_hbm, out_hbm):
            pltpu.sync_copy(tab_hbm.at[pl.ds(d0, rows_g)], stage_v)
            zeros16 = lax.iota(jnp.int32, _NL) * 0
            row_splats = [zeros16 + r for r in range(rows_g)]

            def body(c, _):
                ids = idx_v[pl.ds(c * _NL, _NL)]
                for r in range(rows_g):
                    out_v[r, pl.ds(c * _NL, _NL)] = plsc.load_gather(
                        stage_v, [row_splats[r], ids])
                return _

            lax.fori_loop(0, s // _NL, body, None)
            pltpu.sync_copy(out_v, out_hbm.at[h, pl.ds(d0, rows_g)])

        @pl.when(cid == 0)
        def _():
            pipeline(cos_t_hbm, cos_out)

        @pl.when(cid == 1)
        def _():
            pipeline(sin_t_hbm, sin_out)

    return k


def kernel(qkv, position_ids, cos, sin):
    b, s = position_ids.shape
    n_rows, dim = cos.shape
    cos_ot, sin_ot = _make(b, s, n_rows, dim)(
        position_ids.astype(jnp.int32), cos.T, sin.T)
    return cos_ot.transpose(0, 2, 1), sin_ot.transpose(0, 2, 1)


# speedup vs baseline: 1.7996x; 1.2890x over previous
import jax
import jax.numpy as jnp
from jax import lax
from jax.experimental import pallas as pl
from jax.experimental.pallas import tpu as pltpu
from jax.experimental.pallas import tpu_sc as plsc

_INFO = plsc.get_sparse_core_info()
_NC = _INFO.num_cores
_NS = _INFO.num_subcores
_NL = _INFO.num_lanes


def _make(b, s, n_rows, dim):
    groups = _NS // b
    rows_g = dim // groups
    mesh = plsc.VectorSubcoreMesh(core_axis_name="c", subcore_axis_name="s")

    @pl.kernel(
        mesh=mesh,
        compiler_params=pltpu.CompilerParams(use_tc_tiling_on_sc=True,
                                             needs_layout_passes=False),
        out_type=(
            jax.ShapeDtypeStruct((b, dim, s), jnp.float32),
            jax.ShapeDtypeStruct((b, dim, s), jnp.float32),
        ),
        scratch_types=[
            pltpu.VMEM((s,), jnp.int32),
            pltpu.VMEM((rows_g, n_rows), jnp.float32),
            pltpu.VMEM((rows_g, s), jnp.float32),
        ],
    )
    def k(pos_hbm, cos_t_hbm, sin_t_hbm, cos_out, sin_out,
          idx_v, stage_v, out_v):
        cid = lax.axis_index("c")
        sid = lax.axis_index("s")
        g = sid % groups
        h = sid // groups
        d0 = g * rows_g

        pltpu.sync_copy(pos_hbm.at[h], idx_v)

        def pipeline(tab_hbm, out_hbm):
            pltpu.sync_copy(tab_hbm.at[pl.ds(d0, rows_g)], stage_v)
            zeros16 = lax.iota(jnp.int32, _NL) * 0
            row_splats = [zeros16 + r for r in range(rows_g)]

            @plsc.parallel_loop(0, s // _NL, unroll=8)
            def body(c):
                ids = idx_v[pl.ds(c * _NL, _NL)]
                for r in range(rows_g):
                    out_v[r, pl.ds(c * _NL, _NL)] = plsc.load_gather(
                        stage_v, [row_splats[r], ids])
            pltpu.sync_copy(out_v, out_hbm.at[h, pl.ds(d0, rows_g)])

        @pl.when(cid == 0)
        def _():
            pipeline(cos_t_hbm, cos_out)

        @pl.when(cid == 1)
        def _():
            pipeline(sin_t_hbm, sin_out)

    return k


def kernel(qkv, position_ids, cos, sin):
    b, s = position_ids.shape
    n_rows, dim = cos.shape
    cos_ot, sin_ot = _make(b, s, n_rows, dim)(
        position_ids.astype(jnp.int32), cos.T, sin.T)
    return cos_ot.transpose(0, 2, 1), sin_ot.transpose(0, 2, 1)
